# baseline (speedup 1.0000x reference)
import functools

import jax
import jax.numpy as jnp
from jax import lax
from jax.experimental import pallas as pl
from jax.experimental.pallas import tpu as pltpu
from jax.experimental.pallas import tpu_sc as plsc

_B = 4096
_L = 200
_D = 32
_V = 1000000
_NW = 32
_QUAD = 512
_NFULL = _V // _QUAD
_QPW = 61
_TAIL = _V - _NFULL * _QUAD
_RROWS = (_V + 63) // 64 * 16 + 16
_BLK = 128
_NBLK = _L * (_B // _BLK)
_BPW = _NBLK // _NW

_CP = pltpu.CompilerParams(use_tc_tiling_on_sc=True, needs_layout_passes=False)


def _build_relayout():
  mesh = plsc.VectorSubcoreMesh(core_axis_name="c", subcore_axis_name="s")

  @functools.partial(
      pl.kernel,
      mesh=mesh,
      compiler_params=_CP,
      out_type=jax.ShapeDtypeStruct((_RROWS, 128), jnp.float32),
      scratch_types=[
          pltpu.VMEM((_D, _QUAD), jnp.float32),
          pltpu.VMEM((_D, _QUAD), jnp.float32),
          pltpu.VMEM((128, 128), jnp.float32),
          pltpu.VMEM((128, 128), jnp.float32),
          pltpu.SemaphoreType.DMA,
          pltpu.SemaphoreType.DMA,
          pltpu.SemaphoreType.DMA,
          pltpu.SemaphoreType.DMA,
      ],
  )
  def k(tableT, tailR, tabr, tin0, tin1, trow0, trow1,
        s_i0, s_i1, s_o0, s_o1):
    wid = lax.axis_index("c") * 16 + lax.axis_index("s")
    nq = jnp.where(wid == _NW - 1, _QPW + 1, _QPW)
    qbase = wid * _QPW
    iota16 = lax.iota(jnp.int32, 16)

    def start_in(q, tin, sem):
      pltpu.async_copy(tableT.at[:, pl.ds(q * _QUAD, _QUAD)], tin, sem)

    def drain_in(tin, sem):
      pltpu.make_async_copy(
          tableT.at[:, pl.ds(0, _QUAD)], tin, sem).wait()

    def transpose_quad(tin, trow):
      @plsc.parallel_loop(0, _QUAD, unroll=4)
      def _t(r):
        col = jnp.full((16,), r, jnp.int32)
        row = lax.shift_right_logical(r, 2)
        cb = (r & 3) * _D
        trow[row, pl.ds(cb, 16)] = plsc.load_gather(tin, [iota16, col])
        trow[row, pl.ds(cb + 16, 16)] = plsc.load_gather(
            tin, [iota16 + 16, col])

    def start_out(q, trow, sem):
      pltpu.async_copy(trow, tabr.at[pl.ds(q * 128, 128)], sem)

    def drain_out(trow, sem):
      pltpu.make_async_copy(trow, tabr.at[pl.ds(0, 128)], sem).wait()

    start_in(qbase, tin0, s_i0)

    @pl.loop(0, (_QPW + 1) // 2 + 1)
    def _p1(i):
      ka = i * 2
      kb = ka + 1

      @pl.when(kb < nq)
      def _():
        start_in(qbase + kb, tin1, s_i1)

      @pl.when(ka < nq)
      def _():
        drain_in(tin0, s_i0)

        @pl.when(ka >= 2)
        def _():
          drain_out(trow0, s_o0)

        transpose_quad(tin0, trow0)
        start_out(qbase + ka, trow0, s_o0)

      @pl.when(ka + 2 < nq)
      def _():
        start_in(qbase + ka + 2, tin0, s_i0)

      @pl.when(kb < nq)
      def _():
        drain_in(tin1, s_i1)

        @pl.when(kb >= 2)
        def _():
          drain_out(trow1, s_o1)

        transpose_quad(tin1, trow1)
        start_out(qbase + kb, trow1, s_o1)

    drain_out(trow0, s_o0)
    drain_out(trow1, s_o1)

    @pl.when(wid == _NW - 1)
    def _tail():
      pltpu.sync_copy(tailR, tin0.at[:, pl.ds(0, 128)])

      @plsc.parallel_loop(0, _TAIL, unroll=4)
      def _t(r):
        col = jnp.full((16,), r, jnp.int32)
        row = lax.shift_right_logical(r, 2)
        cb = (r & 3) * _D
        trow0[row, pl.ds(cb, 16)] = plsc.load_gather(tin0, [iota16, col])
        trow0[row, pl.ds(cb + 16, 16)] = plsc.load_gather(
            tin0, [iota16 + 16, col])

      pltpu.sync_copy(trow0.at[pl.ds(0, _TAIL // 4)],
                      tabr.at[pl.ds(_NFULL * 128, _TAIL // 4)])

  return k


def _build_gather():
  mesh = plsc.VectorSubcoreMesh(core_axis_name="c", subcore_axis_name="s")

  @functools.partial(
      pl.kernel,
      mesh=mesh,
      compiler_params=_CP,
      out_type=jax.ShapeDtypeStruct((_L, _D, _B), jnp.float32),
      scratch_types=[
          pltpu.VMEM((_BLK,), jnp.int32),
          pltpu.VMEM((_BLK,), jnp.int32),
          pltpu.VMEM((_BLK,), jnp.int32),
          pltpu.VMEM((_BLK,), jnp.int32),
          pltpu.VMEM((_BLK,), jnp.int32),
          pltpu.VMEM((_BLK,), jnp.int32),
          pltpu.VMEM((_BLK, 128), jnp.float32),
          pltpu.VMEM((_BLK, 128), jnp.float32),
          pltpu.VMEM((_D, _BLK), jnp.float32),
          pltpu.VMEM((_D, _BLK), jnp.float32),
          pltpu.VMEM((_D, 256), jnp.float32),
          pltpu.SemaphoreType.DMA,
          pltpu.SemaphoreType.DMA,
          pltpu.SemaphoreType.DMA,
          pltpu.SemaphoreType.DMA,
          pltpu.SemaphoreType.DMA,
          pltpu.SemaphoreType.DMA,
      ],
  )
  def k(xT, tabr, posT, outT,
        idx0, idx1, idxq0, idxq1, colb0, colb1, g0, g1,
        obuf0, obuf1, posv, si0, si1, sg0, sg1, so0, so1):
    wid = lax.axis_index("c") * 16 + lax.axis_index("s")
    iota16 = lax.iota(jnp.int32, 16)
    pltpu.sync_copy(posT, posv)
    q0 = wid * _BPW
    jblk = _B // _BLK

    def start_idx(n, ibuf, sem):
      q = q0 + n
      l = q // jblk
      j2 = q % jblk
      pltpu.async_copy(xT.at[l, pl.ds(j2 * _BLK, _BLK)], ibuf, sem)

    def drain_idx(ibuf, sem):
      pltpu.make_async_copy(xT.at[0, pl.ds(0, _BLK)], ibuf, sem).wait()

    def prep_fire(ibuf, iq, cb, g, sem):
      for h in range(8):
        v = ibuf[pl.ds(16 * h, 16)]
        iq[pl.ds(16 * h, 16)] = lax.shift_right_logical(v, 2)
        cb[pl.ds(16 * h, 16)] = (v & 3) * _D
      pltpu.async_copy(tabr.at[iq], g, sem)

    def drain_gather(g, sem):
      pltpu.make_async_copy(tabr.at[pl.ds(0, _BLK)], g, sem).wait()

    def vec_block(n, g, cb, obuf):
      q = q0 + n
      l = q // jblk
      pvs = []
      for d in range(_D):
        pvs.append(jnp.full((16,), posv[d, pl.ds(l, 16)][0], jnp.float32))
      for h in range(8):
        rows = iota16 + 16 * h
        cols = cb[pl.ds(16 * h, 16)]
        for d in range(_D):
          vals = plsc.load_gather(g, [rows, cols + d])
          obuf[d, pl.ds(16 * h, 16)] = vals + pvs[d]

    def start_out(n, obuf, sem):
      q = q0 + n
      l = q // jblk
      j2 = q % jblk
      pltpu.async_copy(obuf, outT.at[l, :, pl.ds(j2 * _BLK, _BLK)], sem)

    def drain_out(obuf, sem):
      pltpu.make_async_copy(
          obuf, outT.at[0, :, pl.ds(0, _BLK)], sem).wait()

    start_idx(0, idx0, si0)
    start_idx(1, idx1, si1)
    drain_idx(idx0, si0)
    prep_fire(idx0, idxq0, colb0, g0, sg0)
    drain_idx(idx1, si1)
    prep_fire(idx1, idxq1, colb1, g1, sg1)
    start_idx(2, idx0, si0)
    start_idx(3, idx1, si1)

    @pl.loop(0, _BPW // 2)
    def _p2(j):
      a = j * 2

      drain_gather(g0, sg0)

      @pl.when(j > 0)
      def _():
        drain_out(obuf0, so0)

      vec_block(a, g0, colb0, obuf0)
      start_out(a, obuf0, so0)

      @pl.when(j < _BPW // 2 - 1)
      def _():
        drain_idx(idx0, si0)
        prep_fire(idx0, idxq0, colb0, g0, sg0)

      @pl.when(j < _BPW // 2 - 2)
      def _():
        start_idx(a + 4, idx0, si0)

      drain_gather(g1, sg1)

      @pl.when(j > 0)
      def _():
        drain_out(obuf1, so1)

      vec_block(a + 1, g1, colb1, obuf1)
      start_out(a + 1, obuf1, so1)

      @pl.when(j < _BPW // 2 - 1)
      def _():
        drain_idx(idx1, si1)
        prep_fire(idx1, idxq1, colb1, g1, sg1)

      @pl.when(j < _BPW // 2 - 2)
      def _():
        start_idx(a + 5, idx1, si1)

    drain_out(obuf0, so0)
    drain_out(obuf1, so1)

  return k


_k_relayout = _build_relayout()
_k_gather = _build_gather()


def kernel(x, token_table, pos_table):
  pos_p = jnp.pad(pos_table.T, ((0, 0), (0, 256 - _L)))
  tail_p = jnp.pad(token_table[_NFULL * _QUAD:, :].T,
                   ((0, 0), (0, 128 - _TAIL)))
  tabr = _k_relayout(token_table.T, tail_p)
  outT = _k_gather(x.T, tabr, pos_p)
  return outT.transpose(2, 0, 1)


# speedup vs baseline: 1.1872x; 1.1872x over previous
import functools

import jax
import jax.numpy as jnp
from jax import lax
from jax.experimental import pallas as pl
from jax.experimental.pallas import tpu as pltpu
from jax.experimental.pallas import tpu_sc as plsc

_B = 4096
_L = 200
_D = 32
_NW = 32
_IDXW = 100
_SEQ_PER_CHUNK = 8
_CHUNK_IDX = _SEQ_PER_CHUNK * _L
_CHUNK_ROWS = _CHUNK_IDX // _IDXW
_TOT_IDX = _B * _L
_IDX_ROWS = _TOT_IDX // _IDXW
_ROWS_PER_W = _IDX_ROWS // _NW
_CHUNKS_PER_W = _ROWS_PER_W // _CHUNK_ROWS
_NBODY = _CHUNKS_PER_W // 2


def _build():
  mesh = plsc.VectorSubcoreMesh(core_axis_name="c", subcore_axis_name="s")

  @functools.partial(
      pl.kernel,
      mesh=mesh,
      compiler_params=pltpu.CompilerParams(use_tc_tiling_on_sc=False),
      out_type=jax.ShapeDtypeStruct((_TOT_IDX, _D), jnp.float32),
      scratch_types=[
          pltpu.VMEM((_CHUNK_ROWS, _IDXW), jnp.int32),
          pltpu.VMEM((_CHUNK_ROWS, _IDXW), jnp.int32),
          pltpu.VMEM((_CHUNK_IDX, _D), jnp.float32),
          pltpu.VMEM((_CHUNK_IDX, _D), jnp.float32),
          pltpu.VMEM((_L, _D), jnp.float32),
          pltpu.SemaphoreType.DMA,
          pltpu.SemaphoreType.DMA,
          pltpu.SemaphoreType.DMA,
          pltpu.SemaphoreType.DMA,
          pltpu.SemaphoreType.DMA,
          pltpu.SemaphoreType.DMA,
      ],
  )
  def k(x_hbm, table_hbm, pos_hbm, out_hbm,
        idx0, idx1, rows0, rows1, pos_v,
        si0, si1, sg0, sg1, so0, so1):
    wid = lax.axis_index("c") * 16 + lax.axis_index("s")

    def start_idx(chunk, ibuf, sem):
      base = wid * _ROWS_PER_W + chunk * _CHUNK_ROWS
      pltpu.async_copy(x_hbm.at[pl.ds(base, _CHUNK_ROWS)], ibuf, sem)

    def drain_idx(ibuf, sem):
      pltpu.make_async_copy(
          x_hbm.at[pl.ds(0, _CHUNK_ROWS)], ibuf, sem).wait()

    def fire_gathers(ibuf, rbuf, sem):
      for j in range(_CHUNK_ROWS):
        pltpu.async_copy(
            table_hbm.at[ibuf.at[j]],
            rbuf.at[pl.ds(j * _IDXW, _IDXW)],
            sem,
        )

    def drain_gathers(rbuf, sem):
      pltpu.make_async_copy(
          table_hbm.at[pl.ds(0, _CHUNK_IDX)], rbuf, sem).wait()

    def start_out(chunk, rbuf, sem):
      base = (wid * _ROWS_PER_W + chunk * _CHUNK_ROWS) * _IDXW
      pltpu.async_copy(rbuf, out_hbm.at[pl.ds(base, _CHUNK_IDX)], sem)

    def drain_out(rbuf, sem):
      pltpu.make_async_copy(
          rbuf, out_hbm.at[pl.ds(0, _CHUNK_IDX)], sem).wait()

    def pos_add(rbuf):
      for h in range(2):
        @plsc.parallel_loop(0, _L, unroll=8)
        def _row(r):
          p = pos_v[r, pl.ds(h * 16, 16)]
          for s in range(_SEQ_PER_CHUNK):
            rbuf[s * _L + r, pl.ds(h * 16, 16)] += p

    pltpu.sync_copy(pos_hbm, pos_v)
    start_idx(0, idx0, si0)
    drain_idx(idx0, si0)
    fire_gathers(idx0, rows0, sg0)
    start_idx(1, idx1, si1)

    @pl.loop(0, _NBODY)
    def _body(i):
      c0 = i * 2

      drain_gathers(rows0, sg0)
      drain_idx(idx1, si1)

      @pl.when(i > 0)
      def _():
        drain_out(rows1, so1)

      fire_gathers(idx1, rows1, sg1)

      @pl.when(i < _NBODY - 1)
      def _():
        start_idx(c0 + 2, idx0, si0)

      pos_add(rows0)
      start_out(c0, rows0, so0)

      drain_gathers(rows1, sg1)

      @pl.when(i < _NBODY - 1)
      def _():
        drain_idx(idx0, si0)

      drain_out(rows0, so0)

      @pl.when(i < _NBODY - 1)
      def _():
        fire_gathers(idx0, rows0, sg0)
        start_idx(c0 + 3, idx1, si1)

      pos_add(rows1)
      start_out(c0 + 1, rows1, so1)

    drain_out(rows1, so1)

  return k


_k = _build()


def kernel(x, token_table, pos_table):
  xf = x.reshape(_IDX_ROWS, _IDXW)
  out = _k(xf, token_table, pos_table)
  return out.reshape(_B, _L, _D)
